# baseline (speedup 1.0000x reference)
import functools

import jax
import jax.numpy as jnp
from jax import lax
from jax.experimental import pallas as pl
from jax.experimental.pallas import tpu as pltpu
from jax.experimental.pallas import tpu_sc as plsc

N_NODES = 10000
N_EDGES = 320000
D = 128
L = 16
NC, NS = 2, 16
NW = NC * NS
EPW = N_EDGES // NW
BLK = 2000
NBLK = EPW // BLK
C = 80
CPB = BLK // C
NPAIR = (CPB + 1) // 2

STAGE = -(-N_NODES // NS)
STAGE = ((STAGE + 7) // 8) * 8
LAST_STAGE = N_NODES - STAGE


def _sc_body(z_hbm, row_hbm, col_hbm, out_hbm,
             idx_r, idx_c, rows, cols, out_v, tr, z_sp,
             sem_r0, sem_c0, sem_r1, sem_c1):
    cid = lax.axis_index("c")
    sid = lax.axis_index("s")
    wid = sid * NC + cid
    ebase = pl.multiple_of(wid * EPW, 8)

    lane = lax.broadcasted_iota(jnp.int32, (L,), 0)
    sems = [(sem_r0, sem_c0), (sem_r1, sem_c1)]
    last_off = BLK - C

    zoff = pl.multiple_of(jnp.minimum(sid * STAGE, LAST_STAGE), 8)
    pltpu.sync_copy(z_hbm.at[pl.ds(zoff, STAGE)], z_sp.at[pl.ds(zoff, STAGE)])
    plsc.subcore_barrier()

    def chunk_off(k):
        return pl.multiple_of(jnp.minimum(k * C, last_off), 8)

    def start_gather(b, k):
        off = chunk_off(k)
        sr, sc_ = sems[b]
        pltpu.make_async_copy(
            z_sp.at[idx_r.at[pl.ds(off, C)]], rows.at[b], sr).start()
        pltpu.make_async_copy(
            z_sp.at[idx_c.at[pl.ds(off, C)]], cols.at[b], sc_).start()

    def wait_gather(b):
        sr, sc_ = sems[b]
        pltpu.make_async_copy(z_sp.at[pl.ds(0, C)], rows.at[b], sr).wait()
        pltpu.make_async_copy(z_sp.at[pl.ds(0, C)], cols.at[b], sc_).wait()

    tr_stride = L + 1
    tr_base = lane * tr_stride

    def compute(b, k):
        obase = chunk_off(k)

        def body16(i, carry):
            for e2 in range(L):
                e = i * L + e2
                acc = rows[b, e, pl.ds(0, L)] * cols[b, e, pl.ds(0, L)]
                for j in range(1, D // L):
                    acc = acc + (rows[b, e, pl.ds(j * L, L)]
                                 * cols[b, e, pl.ds(j * L, L)])
                tr[pl.ds(e2 * tr_stride, L)] = acc
            vec = plsc.load_gather(tr, [tr_base])
            for k2 in range(1, L):
                vec = vec + plsc.load_gather(tr, [tr_base + k2])
            out_v[pl.ds(obase + i * L, L)] = 1.0 / (1.0 + jnp.exp(-vec))
            return carry

        lax.fori_loop(0, C // L, body16, 0, unroll=False)

    def block(blk, carry):
        bbase = pl.multiple_of(ebase + blk * BLK, 8)
        pltpu.sync_copy(row_hbm.at[pl.ds(bbase, BLK)], idx_r)
        pltpu.sync_copy(col_hbm.at[pl.ds(bbase, BLK)], idx_c)

        start_gather(0, jnp.int32(0))

        def pair(p, carry2):
            k0 = 2 * p
            start_gather(1, k0 + 1)
            wait_gather(0)
            compute(0, k0)
            start_gather(0, k0 + 2)
            wait_gather(1)
            compute(1, k0 + 1)
            return carry2

        lax.fori_loop(0, NPAIR, pair, 0, unroll=False)
        wait_gather(0)

        pltpu.sync_copy(out_v, out_hbm.at[pl.ds(bbase, BLK)])
        return carry

    lax.fori_loop(0, NBLK, block, 0, unroll=False)


@functools.partial(
    pl.kernel,
    out_type=jax.ShapeDtypeStruct((N_EDGES,), jnp.float32),
    mesh=plsc.VectorSubcoreMesh(core_axis_name="c", subcore_axis_name="s"),
    compiler_params=pltpu.CompilerParams(needs_layout_passes=False),
    scratch_types=[
        pltpu.VMEM((BLK,), jnp.int32),
        pltpu.VMEM((BLK,), jnp.int32),
        pltpu.VMEM((2, C, D), jnp.float32),
        pltpu.VMEM((2, C, D), jnp.float32),
        pltpu.VMEM((BLK,), jnp.float32),
        pltpu.VMEM((L * (L + 1) + 8,), jnp.float32),
        pltpu.VMEM_SHARED((N_NODES, D), jnp.float32),
        pltpu.SemaphoreType.DMA,
        pltpu.SemaphoreType.DMA,
        pltpu.SemaphoreType.DMA,
        pltpu.SemaphoreType.DMA,
    ],
)
def _edge_probs_sc(z_hbm, row_hbm, col_hbm, out_hbm, *scratch):
    _sc_body(z_hbm, row_hbm, col_hbm, out_hbm, *scratch)


def kernel(z, edge_index):
    row = edge_index[0].astype(jnp.int32)
    col = edge_index[1].astype(jnp.int32)
    return _edge_probs_sc(z, row, col)
